# baseline (speedup 1.0000x reference)
import functools

import jax
import jax.numpy as jnp
from jax import lax
from jax.experimental import pallas as pl
from jax.experimental.pallas import tpu as pltpu
from jax.experimental.pallas import tpu_sc as plsc

N = 10000
D = 128
E = 320000
NC = 2
NS = 16
NW = NC * NS
L = 16
EPW = E // NW
CH = 80
NG = CH // L
NCHUNK = EPW // CH
RPT = 624
TAIL = N - NS * RPT

_LANE = lambda: lax.iota(jnp.int32, L)

_mesh = plsc.VectorSubcoreMesh(core_axis_name="c", subcore_axis_name="s")
_params = pltpu.CompilerParams(needs_layout_passes=False)


def _vgather(v, idx):
    return v.at[idx].get(mode="promise_in_bounds")


def _seg_combine(keys, vals, op):
    sk, sv = plsc.sort_key_val(keys, vals)
    lane = _LANE()
    for sh in (1, 2, 4, 8):
        idx = jnp.maximum(lane - sh, 0)
        k2 = _vgather(sk, idx)
        v2 = _vgather(sv, idx)
        take = (k2 == sk) & (lane >= sh)
        sv = jnp.where(take, op(sv, v2), sv)
    nxt = _vgather(sk, jnp.minimum(lane + 1, L - 1))
    is_last = (lane == L - 1) | (nxt != sk)
    return sk, sv, is_last


@functools.partial(
    pl.kernel,
    mesh=_mesh,
    compiler_params=_params,
    out_type=[
        jax.ShapeDtypeStruct((E,), jnp.float32),
        jax.ShapeDtypeStruct((NW, 1, N), jnp.float32),
    ],
    scratch_types=[
        pltpu.VMEM((EPW,), jnp.int32),
        pltpu.VMEM((EPW,), jnp.int32),
        pltpu.VMEM((EPW,), jnp.float32),
        pltpu.VMEM((CH, D), jnp.float32),
        pltpu.VMEM((CH, D), jnp.float32),
        pltpu.VMEM((CH, D), jnp.float32),
        pltpu.VMEM((CH, D), jnp.float32),
        pltpu.VMEM((N,), jnp.float32),
        pltpu.SemaphoreType.DMA,
        pltpu.SemaphoreType.DMA,
        pltpu.SemaphoreType.DMA,
        pltpu.SemaphoreType.DMA,
    ],
)
def _edge_dot_max(emb, srcs, dsts, norm_out, mpart_out,
                  siall, diall, nall, sr0, sr1, dr0, dr1, mloc,
                  ss0, ss1, sd0, sd1):
    wid = lax.axis_index("s") * NC + lax.axis_index("c")
    lane = _LANE()
    SR = (sr0, sr1)
    DR = (dr0, dr1)
    SS = (ss0, ss1)
    SD = (sd0, sd1)

    def init_body(i, c):
        mloc[pl.ds(i * L, L)] = jnp.full((L,), -jnp.inf, jnp.float32)
        return c
    lax.fori_loop(0, N // L, init_body, 0)

    base0 = wid * EPW
    pltpu.sync_copy(srcs.at[pl.ds(base0, EPW)], siall)
    pltpu.sync_copy(dsts.at[pl.ds(base0, EPW)], diall)

    def issue(ci, b):
        sl = pl.ds(ci * CH, CH)
        pltpu.async_copy(emb.at[siall.at[sl]], SR[b], SS[b])
        pltpu.async_copy(emb.at[diall.at[sl]], DR[b], SD[b])

    def wait(ci, b):
        sl = pl.ds(ci * CH, CH)
        pltpu.make_async_copy(emb.at[siall.at[sl]], SR[b], SS[b]).wait()
        pltpu.make_async_copy(emb.at[diall.at[sl]], DR[b], SD[b]).wait()

    def compute(ci, b):
        sr = SR[b]
        dr = DR[b]

        def group_body(g, c2):
            normv = jnp.zeros((L,), jnp.float32)
            for k in range(L):
                acc = sr[g * L + k, pl.ds(0, L)] * dr[g * L + k, pl.ds(0, L)]
                for j in range(1, D // L):
                    acc = acc + (sr[g * L + k, pl.ds(j * L, L)]
                                 * dr[g * L + k, pl.ds(j * L, L)])
                for sh in (1, 2, 4, 8):
                    acc = acc + _vgather(acc, lane ^ sh)
                normv = jnp.where(lane == k, acc, normv)
            nall[pl.ds(ci * CH + g * L, L)] = normv
            dv = diall[pl.ds(ci * CH + g * L, L)]
            sk, sv, is_last = _seg_combine(dv, normv, jnp.maximum)
            cur = plsc.load_gather(mloc, [sk], mask=is_last)
            plsc.store_scatter(mloc, [sk], jnp.maximum(cur, sv), mask=is_last)
            return c2
        lax.fori_loop(0, NG, group_body, 0)

    issue(0, 0)

    def pair_body(i, c):
        ci0 = 2 * i
        issue(ci0 + 1, 1)
        wait(ci0, 0)
        compute(ci0, 0)
        issue(ci0 + 2, 0)
        wait(ci0 + 1, 1)
        compute(ci0 + 1, 1)
        return c
    lax.fori_loop(0, (NCHUNK - 1) // 2, pair_body, 0)

    wait(NCHUNK - 1, 0)
    compute(NCHUNK - 1, 0)

    pltpu.sync_copy(nall, norm_out.at[pl.ds(base0, EPW)])
    pltpu.sync_copy(mloc, mpart_out.at[wid, 0])


def _max_reduce_body(mp_ref, m_ref):
    m = jnp.max(mp_ref[:, 0, :], axis=0)
    m_ref[...] = jnp.where(jnp.isfinite(m), m, 0.0)


@functools.partial(
    pl.kernel,
    mesh=_mesh,
    compiler_params=_params,
    out_type=[
        jax.ShapeDtypeStruct((NC, N, D), jnp.float32),
        jax.ShapeDtypeStruct((NW, 1, N), jnp.float32),
    ],
    scratch_types=[
        pltpu.VMEM((CH,), jnp.int32),
        pltpu.VMEM((CH,), jnp.int32),
        pltpu.VMEM((CH,), jnp.int32),
        pltpu.VMEM((CH,), jnp.int32),
        pltpu.VMEM((CH,), jnp.float32),
        pltpu.VMEM((CH,), jnp.float32),
        pltpu.VMEM((CH, D), jnp.float32),
        pltpu.VMEM((CH, D), jnp.float32),
        pltpu.VMEM((N,), jnp.float32),
        pltpu.VMEM((N,), jnp.float32),
        pltpu.VMEM_SHARED((N, D), jnp.float32),
        pltpu.SemaphoreType.DMA,
        pltpu.SemaphoreType.DMA,
    ],
)
def _edge_softmax_aggregate(emb, srcs, dsts, norm_in, m_in, zeros_in,
                            npart_out, dpart_out,
                            si0, si1, di0, di1, nb0, nb1, sr0, sr1,
                            mloc, dloc, acc, ss0, ss1):
    cid = lax.axis_index("c")
    sid = lax.axis_index("s")
    wid = sid * NC + cid
    SI = (si0, si1)
    DI = (di0, di1)
    NB = (nb0, nb1)
    SR = (sr0, sr1)
    SS = (ss0, ss1)

    pltpu.sync_copy(m_in, mloc)

    def zinit_body(i, c):
        dloc[pl.ds(i * L, L)] = jnp.zeros((L,), jnp.float32)
        return c
    lax.fori_loop(0, N // L, zinit_body, 0)

    pltpu.sync_copy(zeros_in, acc.at[pl.ds(sid * RPT, RPT)])
    @pl.when(sid == NS - 1)
    def _zero_tail():
        pltpu.sync_copy(zeros_in.at[pl.ds(0, TAIL)],
                        acc.at[pl.ds(NS * RPT, TAIL)])
    plsc.subcore_barrier()

    base0 = wid * EPW

    def issue(ci, b):
        base = base0 + ci * CH
        pltpu.sync_copy(srcs.at[pl.ds(base, CH)], SI[b])
        pltpu.sync_copy(dsts.at[pl.ds(base, CH)], DI[b])
        pltpu.sync_copy(norm_in.at[pl.ds(base, CH)], NB[b])
        pltpu.async_copy(emb.at[SI[b]], SR[b], SS[b])

    def wait(ci, b):
        pltpu.make_async_copy(emb.at[SI[b]], SR[b], SS[b]).wait()

    def compute(ci, b):
        sr = SR[b]
        di = DI[b]
        nb = NB[b]

        def group_body(g, c2):
            dv = di[pl.ds(g * L, L)]
            mv = plsc.load_gather(mloc, [dv])
            ev = jnp.exp(nb[pl.ds(g * L, L)] - mv)
            sk, sv, is_last = _seg_combine(dv, ev, jnp.add)
            cur = plsc.load_gather(dloc, [sk], mask=is_last)
            plsc.store_scatter(dloc, [sk], cur + sv, mask=is_last)
            for k in range(L):
                w = ev[k]
                for j in range(D // L):
                    sr[g * L + k, pl.ds(j * L, L)] = (
                        sr[g * L + k, pl.ds(j * L, L)] * w)
            return c2
        lax.fori_loop(0, NG, group_body, 0)
        pltpu.sync_copy(sr, acc.at[di], add=True)

    issue(0, 0)

    def pair_body(i, c):
        ci0 = 2 * i
        issue(ci0 + 1, 1)
        wait(ci0, 0)
        compute(ci0, 0)
        issue(ci0 + 2, 0)
        wait(ci0 + 1, 1)
        compute(ci0 + 1, 1)
        return c
    lax.fori_loop(0, (NCHUNK - 1) // 2, pair_body, 0)

    wait(NCHUNK - 1, 0)
    compute(NCHUNK - 1, 0)

    plsc.subcore_barrier()

    pltpu.sync_copy(acc.at[pl.ds(sid * RPT, RPT)],
                    npart_out.at[cid, pl.ds(sid * RPT, RPT)])
    @pl.when(sid == NS - 1)
    def _dump_tail():
        pltpu.sync_copy(acc.at[pl.ds(NS * RPT, TAIL)],
                        npart_out.at[cid, pl.ds(NS * RPT, TAIL)])
    pltpu.sync_copy(dloc, dpart_out.at[wid, 0])


def _finalize_body(np_ref, dp_ref, w_ref, g_ref, b_ref, o_ref):
    neigh = np_ref[0] + np_ref[1]
    denom = jnp.sum(dp_ref[:, 0, :], axis=0)
    denom = jnp.where(denom > 0.0, denom, 1.0)
    neigh = neigh / denom[:, None]
    h = jnp.dot(neigh, w_ref[...], preferred_element_type=jnp.float32)
    mean = jnp.mean(h, axis=0)
    var = jnp.mean(jnp.square(h - mean), axis=0)
    o_ref[...] = jnp.tanh((h - mean) * lax.rsqrt(var + 1e-5) * g_ref[...]
                          + b_ref[...])


def kernel(ent_emb, edge_index, neigh_w, bn_gamma, bn_beta):
    src = edge_index[0]
    dst = edge_index[1]

    norm, m_part = _edge_dot_max(ent_emb, src, dst)

    m = pl.pallas_call(
        _max_reduce_body,
        out_shape=jax.ShapeDtypeStruct((N,), jnp.float32),
    )(m_part)

    zeros = jnp.zeros((RPT, D), jnp.float32)
    npart, dpart = _edge_softmax_aggregate(ent_emb, src, dst, norm, m, zeros)

    out = pl.pallas_call(
        _finalize_body,
        out_shape=jax.ShapeDtypeStruct((N, D), jnp.float32),
    )(npart, dpart, neigh_w, bn_gamma, bn_beta)
    return out


# speedup vs baseline: 1.0899x; 1.0899x over previous
import functools

import jax
import jax.numpy as jnp
from jax import lax
from jax.experimental import pallas as pl
from jax.experimental.pallas import tpu as pltpu
from jax.experimental.pallas import tpu_sc as plsc

N = 10000
D = 128
E = 320000
NC = 2
NS = 16
NW = NC * NS
L = 16
EPW = E // NW
CH = 80
NG = CH // L
NCHUNK = EPW // CH
RPT = 624
TAIL = N - NS * RPT

_LANE = lambda: lax.iota(jnp.int32, L)

_mesh = plsc.VectorSubcoreMesh(core_axis_name="c", subcore_axis_name="s")
_params = pltpu.CompilerParams(needs_layout_passes=False)


def _vgather(v, idx):
    return v.at[idx].get(mode="promise_in_bounds")


def _seg_combine(keys, vals, op):
    sk, sv = plsc.sort_key_val(keys, vals)
    lane = _LANE()
    for sh in (1, 2, 4, 8):
        idx = jnp.maximum(lane - sh, 0)
        k2 = _vgather(sk, idx)
        v2 = _vgather(sv, idx)
        take = (k2 == sk) & (lane >= sh)
        sv = jnp.where(take, op(sv, v2), sv)
    nxt = _vgather(sk, jnp.minimum(lane + 1, L - 1))
    is_last = (lane == L - 1) | (nxt != sk)
    return sk, sv, is_last


@functools.partial(
    pl.kernel,
    mesh=_mesh,
    compiler_params=_params,
    out_type=[
        jax.ShapeDtypeStruct((E,), jnp.float32),
        jax.ShapeDtypeStruct((NW, 1, N), jnp.float32),
    ],
    scratch_types=[
        pltpu.VMEM((EPW,), jnp.int32),
        pltpu.VMEM((EPW,), jnp.int32),
        pltpu.VMEM((EPW,), jnp.float32),
        pltpu.VMEM((CH, D), jnp.float32),
        pltpu.VMEM((CH, D), jnp.float32),
        pltpu.VMEM((CH, D), jnp.float32),
        pltpu.VMEM((CH, D), jnp.float32),
        pltpu.VMEM((CH, D), jnp.float32),
        pltpu.VMEM((CH, D), jnp.float32),
        pltpu.VMEM((N,), jnp.float32),
        pltpu.VMEM((L * L,), jnp.float32),
        pltpu.SemaphoreType.DMA,
        pltpu.SemaphoreType.DMA,
        pltpu.SemaphoreType.DMA,
        pltpu.SemaphoreType.DMA,
        pltpu.SemaphoreType.DMA,
        pltpu.SemaphoreType.DMA,
    ],
)
def _edge_dot_max(emb, srcs, dsts, norm_out, mpart_out,
                  siall, diall, nall, sr0, sr1, sr2, dr0, dr1, dr2, mloc, pbuf,
                  ss0, ss1, ss2, sd0, sd1, sd2):
    wid = lax.axis_index("s") * NC + lax.axis_index("c")
    lane = _LANE()
    SR = (sr0, sr1, sr2)
    DR = (dr0, dr1, dr2)
    SS = (ss0, ss1, ss2)
    SD = (sd0, sd1, sd2)

    def init_body(i, c):
        mloc[pl.ds(i * L, L)] = jnp.full((L,), -jnp.inf, jnp.float32)
        return c
    lax.fori_loop(0, N // L, init_body, 0)

    base0 = wid * EPW
    pltpu.sync_copy(srcs.at[pl.ds(base0, EPW)], siall)
    pltpu.sync_copy(dsts.at[pl.ds(base0, EPW)], diall)

    def issue(ci, b):
        sl = pl.ds(ci * CH, CH)
        pltpu.async_copy(emb.at[siall.at[sl]], SR[b], SS[b])
        pltpu.async_copy(emb.at[diall.at[sl]], DR[b], SD[b])

    def wait(ci, b):
        sl = pl.ds(ci * CH, CH)
        pltpu.make_async_copy(emb.at[siall.at[sl]], SR[b], SS[b]).wait()
        pltpu.make_async_copy(emb.at[diall.at[sl]], DR[b], SD[b]).wait()

    def compute(ci, b):
        sr = SR[b]
        dr = DR[b]

        def group_body(g, c2):
            for k in range(L):
                acc = sr[g * L + k, pl.ds(0, L)] * dr[g * L + k, pl.ds(0, L)]
                for j in range(1, D // L):
                    acc = acc + (sr[g * L + k, pl.ds(j * L, L)]
                                 * dr[g * L + k, pl.ds(j * L, L)])
                pbuf[pl.ds(k * L, L)] = acc
            normv = jnp.zeros((L,), jnp.float32)
            for l in range(L):
                flat = lane * L + l
                normv = normv + plsc.load_gather(pbuf, [flat])
            nall[pl.ds(ci * CH + g * L, L)] = normv
            dv = diall[pl.ds(ci * CH + g * L, L)]
            sk, sv, is_last = _seg_combine(dv, normv, jnp.maximum)
            cur = plsc.load_gather(mloc, [sk], mask=is_last)
            plsc.store_scatter(mloc, [sk], jnp.maximum(cur, sv), mask=is_last)
            return c2
        lax.fori_loop(0, NG, group_body, 0)

    issue(0, 0)
    issue(1, 1)

    def tri_body(i, c):
        ci0 = 3 * i
        for b in range(3):
            issue(ci0 + b + 2, (b + 2) % 3)
            wait(ci0 + b, b)
            compute(ci0 + b, b)
        return c
    lax.fori_loop(0, (NCHUNK - 2) // 3, tri_body, 0)

    wait(NCHUNK - 2, 0)
    compute(NCHUNK - 2, 0)
    wait(NCHUNK - 1, 1)
    compute(NCHUNK - 1, 1)

    pltpu.sync_copy(nall, norm_out.at[pl.ds(base0, EPW)])
    pltpu.sync_copy(mloc, mpart_out.at[wid, 0])


def _max_reduce_body(mp_ref, m_ref):
    m = jnp.max(mp_ref[:, 0, :], axis=0)
    m_ref[...] = jnp.where(jnp.isfinite(m), m, 0.0)


@functools.partial(
    pl.kernel,
    mesh=_mesh,
    compiler_params=_params,
    out_type=[
        jax.ShapeDtypeStruct((NC, N, D), jnp.float32),
        jax.ShapeDtypeStruct((NW, 1, N), jnp.float32),
    ],
    scratch_types=[
        pltpu.VMEM((CH,), jnp.int32),
        pltpu.VMEM((CH,), jnp.int32),
        pltpu.VMEM((CH,), jnp.int32),
        pltpu.VMEM((CH,), jnp.int32),
        pltpu.VMEM((CH,), jnp.float32),
        pltpu.VMEM((CH,), jnp.float32),
        pltpu.VMEM((CH, D), jnp.float32),
        pltpu.VMEM((CH, D), jnp.float32),
        pltpu.VMEM((N,), jnp.float32),
        pltpu.VMEM((N,), jnp.float32),
        pltpu.VMEM_SHARED((N, D), jnp.float32),
        pltpu.SemaphoreType.DMA,
        pltpu.SemaphoreType.DMA,
        pltpu.SemaphoreType.DMA,
        pltpu.SemaphoreType.DMA,
    ],
)
def _edge_softmax_aggregate(emb, srcs, dsts, norm_in, m_in, zeros_in,
                            npart_out, dpart_out,
                            si0, si1, di0, di1, nb0, nb1, sr0, sr1,
                            mloc, dloc, acc, ss0, ss1, sc0, sc1):
    cid = lax.axis_index("c")
    sid = lax.axis_index("s")
    wid = sid * NC + cid
    SI = (si0, si1)
    DI = (di0, di1)
    NB = (nb0, nb1)
    SR = (sr0, sr1)
    SS = (ss0, ss1)
    SC = (sc0, sc1)

    pltpu.sync_copy(m_in, mloc)

    def zinit_body(i, c):
        dloc[pl.ds(i * L, L)] = jnp.zeros((L,), jnp.float32)
        return c
    lax.fori_loop(0, N // L, zinit_body, 0)

    pltpu.sync_copy(zeros_in, acc.at[pl.ds(sid * RPT, RPT)])
    @pl.when(sid == NS - 1)
    def _zero_tail():
        pltpu.sync_copy(zeros_in.at[pl.ds(0, TAIL)],
                        acc.at[pl.ds(NS * RPT, TAIL)])
    plsc.subcore_barrier()

    base0 = wid * EPW

    def issue(ci, b):
        base = base0 + ci * CH
        pltpu.sync_copy(srcs.at[pl.ds(base, CH)], SI[b])
        pltpu.sync_copy(dsts.at[pl.ds(base, CH)], DI[b])
        pltpu.sync_copy(norm_in.at[pl.ds(base, CH)], NB[b])
        pltpu.async_copy(emb.at[SI[b]], SR[b], SS[b])

    def wait(ci, b):
        pltpu.make_async_copy(emb.at[SI[b]], SR[b], SS[b]).wait()

    def compute(ci, b):
        sr = SR[b]
        di = DI[b]
        nb = NB[b]

        def group_body(g, c2):
            dv = di[pl.ds(g * L, L)]
            mv = plsc.load_gather(mloc, [dv])
            ev = jnp.exp(nb[pl.ds(g * L, L)] - mv)
            sk, sv, is_last = _seg_combine(dv, ev, jnp.add)
            cur = plsc.load_gather(dloc, [sk], mask=is_last)
            plsc.store_scatter(dloc, [sk], cur + sv, mask=is_last)
            for k in range(L):
                w = ev[k]
                for j in range(D // L):
                    sr[g * L + k, pl.ds(j * L, L)] = (
                        sr[g * L + k, pl.ds(j * L, L)] * w)
            return c2
        lax.fori_loop(0, NG, group_body, 0)
        pltpu.async_copy(sr, acc.at[di], SC[b], add=True)

    def wait_scatter(b):
        pltpu.make_async_copy(SR[b], acc.at[DI[b]], SC[b]).wait()

    issue(0, 0)
    issue(1, 1)

    def pair_body(i, c):
        ci0 = 2 * i
        wait(ci0, 0)
        compute(ci0, 0)
        wait(ci0 + 1, 1)
        compute(ci0 + 1, 1)
        wait_scatter(0)
        issue(ci0 + 2, 0)
        wait_scatter(1)
        issue(ci0 + 3, 1)
        return c
    lax.fori_loop(0, (NCHUNK - 3) // 2, pair_body, 0)

    wait(NCHUNK - 3, 0)
    compute(NCHUNK - 3, 0)
    wait_scatter(0)
    issue(NCHUNK - 1, 0)
    wait(NCHUNK - 2, 1)
    compute(NCHUNK - 2, 1)
    wait_scatter(1)
    wait(NCHUNK - 1, 0)
    compute(NCHUNK - 1, 0)
    wait_scatter(0)

    plsc.subcore_barrier()

    pltpu.sync_copy(acc.at[pl.ds(sid * RPT, RPT)],
                    npart_out.at[cid, pl.ds(sid * RPT, RPT)])
    @pl.when(sid == NS - 1)
    def _dump_tail():
        pltpu.sync_copy(acc.at[pl.ds(NS * RPT, TAIL)],
                        npart_out.at[cid, pl.ds(NS * RPT, TAIL)])
    pltpu.sync_copy(dloc, dpart_out.at[wid, 0])


def _finalize_body(np_ref, dp_ref, w_ref, g_ref, b_ref, o_ref):
    neigh = np_ref[0] + np_ref[1]
    denom = jnp.sum(dp_ref[:, 0, :], axis=0)
    denom = jnp.where(denom > 0.0, denom, 1.0)
    neigh = neigh / denom[:, None]
    h = jnp.dot(neigh, w_ref[...], preferred_element_type=jnp.float32)
    mean = jnp.mean(h, axis=0)
    var = jnp.mean(jnp.square(h - mean), axis=0)
    o_ref[...] = jnp.tanh((h - mean) * lax.rsqrt(var + 1e-5) * g_ref[...]
                          + b_ref[...])


def kernel(ent_emb, edge_index, neigh_w, bn_gamma, bn_beta):
    src = edge_index[0]
    dst = edge_index[1]

    norm, m_part = _edge_dot_max(ent_emb, src, dst)

    m = pl.pallas_call(
        _max_reduce_body,
        out_shape=jax.ShapeDtypeStruct((N,), jnp.float32),
    )(m_part)

    zeros = jnp.zeros((RPT, D), jnp.float32)
    npart, dpart = _edge_softmax_aggregate(ent_emb, src, dst, norm, m, zeros)

    out = pl.pallas_call(
        _finalize_body,
        out_shape=jax.ShapeDtypeStruct((N, D), jnp.float32),
    )(npart, dpart, neigh_w, bn_gamma, bn_beta)
    return out
